# baseline (speedup 1.0000x reference)
import functools

import jax
import jax.numpy as jnp
from jax import lax
from jax.experimental import pallas as pl
from jax.experimental.layout import Layout, with_layout_constraint
from jax.experimental.pallas import tpu as pltpu
from jax.experimental.pallas import tpu_sc as plsc

_N_DENSE = 100
_N_SAMPLES = 20
_N_CHANNELS = 1000
_BATCH = 1024
_ROW_W = _N_DENSE + _N_DENSE * _N_SAMPLES
_SLAB = _N_CHANNELS * _N_SAMPLES

_NC = 2
_NS = 16
_NW = _NC * _NS
_ROWS_PER_W = _BATCH // _NW
_CHUNKS = _N_DENSE * _N_SAMPLES // 16
_NBLK = _SLAB // 128 + 1
_TAIL = _SLAB - (_NBLK - 1) * 128


def _scatter_body(inp_hbm, scr_hbm, out_hbm, inp_v, cidx0, cidx1, qv, rv,
                  sem0, sem1, semr, semw0, semw1):
    c = lax.axis_index("c")
    s = lax.axis_index("s")
    wid = c * _NS + s
    row0 = wid * _ROWS_PER_W

    zvec = jnp.zeros((16,), jnp.float32)
    iota = lax.iota(jnp.int32, 16)
    zvi = iota * 0

    for t in range(5):
        lo = t * 16
        bq = lo // _N_SAMPLES
        cross = (bq + 1) * _N_SAMPLES - lo
        qt = bq + jnp.where(iota >= cross, 1, 0)
        qv[pl.ds(lo, 16)] = qt
        rv[pl.ds(lo, 16)] = (lo + iota) - qt * _N_SAMPLES

    def _rep(j, carry):
        for t in range(5):
            src = pl.ds(t * 16, 16)
            dst = pl.ds(j * 80 + t * 16, 16)
            qv[dst] = qv[src] + j * 4
            rv[dst] = rv[src]
        return carry

    lax.fori_loop(1, _CHUNKS // 5, _rep, 0)

    def _phase1(acc0, acc1):
        def _zero(i, carry):
            dst = pl.ds(i * 16, 16)
            acc0[dst] = zvec
            acc1[dst] = zvec
            return carry

        lax.fori_loop(0, _SLAB // 16, _zero, 0)

        def _pair(p, carry):
            b = row0 + 2 * p
            pltpu.sync_copy(inp_hbm.at[pl.ds(b, 2)], inp_v)
            for r in (0, 1):
                acc = acc0 if r == 0 else acc1
                cidx = cidx0 if r == 0 else cidx1
                sem = sem0 if r == 0 else sem1
                rsp = zvi + r

                @pl.when(p > 0)
                def _():
                    pltpu.make_async_copy(
                        acc, scr_hbm.at[pl.ds(0, _SLAB)], sem).wait()

                    def _clear(k, cc):
                        ds16 = pl.ds(k * 16, 16)
                        fidx = (plsc.load_gather(cidx, [qv[ds16]])
                                + rv[ds16])
                        plsc.store_scatter(acc, [fidx], zvec)
                        return cc

                    lax.fori_loop(0, _CHUNKS, _clear, 0)

                for off in (0, 16, 32, 48, 64, 80, 96):
                    cidx[pl.ds(off, 16)] = (
                        plsc.load_gather(inp_v, [rsp, iota + off])
                        .astype(jnp.int32) * _N_SAMPLES)

                def _accum(k, cc):
                    ds16 = pl.ds(k * 16, 16)
                    fidx = (plsc.load_gather(cidx, [qv[ds16]])
                            + rv[ds16])
                    x = plsc.load_gather(
                        inp_v, [rsp, iota + (_N_DENSE + k * 16)])
                    plsc.addupdate_scatter(acc, [fidx], x)
                    return cc

                lax.fori_loop(0, _CHUNKS, _accum, 0)

                pltpu.async_copy(
                    acc, scr_hbm.at[pl.ds((b + r) * _SLAB, _SLAB)], sem)
            return carry

        lax.fori_loop(0, _ROWS_PER_W // 2, _pair, 0)

        pltpu.make_async_copy(acc0, scr_hbm.at[pl.ds(0, _SLAB)],
                              sem0).wait()
        pltpu.make_async_copy(acc1, scr_hbm.at[pl.ds(0, _SLAB)],
                              sem1).wait()

    pl.run_scoped(_phase1,
                  pltpu.VMEM((_SLAB,), jnp.float32),
                  pltpu.VMEM((_SLAB,), jnp.float32))

    plsc.subcore_barrier()

    iota128 = iota * 128

    def _phase2(in_a, in_b, out_a, out_b):
        def _unit(f, j, in_st, out_st, semw):
            blk = s * 10 + j
            bbase = (c * 4 + f) * 128
            u = f * 10 + j

            @pl.when(blk < _NBLK)
            def _():
                @pl.when(u >= 2)
                def _():
                    @pl.when(jnp.logical_or(j > 0, s < _NS - 1))
                    def _():
                        pltpu.make_async_copy(
                            out_st,
                            out_hbm.at[pl.ds(0, 128), pl.ds(0, 1), :],
                            semw).wait()

                    @pl.when(jnp.logical_and(j == 0, s == _NS - 1))
                    def _():
                        pltpu.make_async_copy(
                            out_st.at[pl.ds(0, _TAIL)],
                            out_hbm.at[pl.ds(0, _TAIL), pl.ds(0, 1), :],
                            semw).wait()

                base = (bbase * _SLAB) + blk * 128

                def _rd(i, cc):
                    for k in range(16):
                        pltpu.async_copy(
                            scr_hbm.at[pl.ds(
                                base + (i * 16 + k) * _SLAB, 128)],
                            in_st.at[pl.ds((i * 16 + k) * 128, 128)],
                            semr)
                    for k in range(16):
                        pltpu.make_async_copy(
                            scr_hbm.at[pl.ds(0, 128)],
                            in_st.at[pl.ds(0, 128)], semr).wait()
                    return cc

                lax.fori_loop(0, 8, _rd, 0)

                def _tr(jo, cc):
                    for ji in range(8):
                        x = plsc.load_gather(
                            in_st, [iota128 + (ji * 2048 + jo)])
                        plsc.store_scatter(
                            out_st, [zvi + jo, zvi, iota + ji * 16], x)
                    return cc

                lax.fori_loop(0, 128, _tr, 0)

                @pl.when(blk < _NBLK - 1)
                def _():
                    pltpu.async_copy(
                        out_st,
                        out_hbm.at[pl.ds(blk * 128, 128),
                                   pl.ds(c * 4 + f, 1), :],
                        semw)

                @pl.when(blk == _NBLK - 1)
                def _():
                    pltpu.async_copy(
                        out_st.at[pl.ds(0, _TAIL)],
                        out_hbm.at[pl.ds((_NBLK - 1) * 128, _TAIL),
                                   pl.ds(c * 4 + f, 1), :],
                        semw)

        def _fbody(f, cc):
            def _jbody(j, cc2):
                @pl.when(lax.bitwise_and(j, 1) == 0)
                def _():
                    _unit(f, j, in_a, out_a, semw0)

                @pl.when(lax.bitwise_and(j, 1) == 1)
                def _():
                    _unit(f, j, in_b, out_b, semw1)
                return cc2

            return lax.fori_loop(0, 10, _jbody, cc)

        lax.fori_loop(0, 4, _fbody, 0)

        @pl.when(s < 15)
        def _():
            pltpu.make_async_copy(
                out_a, out_hbm.at[pl.ds(0, 128), pl.ds(0, 1), :],
                semw0).wait()
            pltpu.make_async_copy(
                out_b, out_hbm.at[pl.ds(0, 128), pl.ds(0, 1), :],
                semw1).wait()

        @pl.when(s == 15)
        def _():
            pltpu.make_async_copy(
                out_a.at[pl.ds(0, _TAIL)],
                out_hbm.at[pl.ds(0, _TAIL), pl.ds(0, 1), :],
                semw0).wait()
            pltpu.make_async_copy(
                out_b, out_hbm.at[pl.ds(0, 128), pl.ds(0, 1), :],
                semw1).wait()

    pl.run_scoped(_phase2,
                  pltpu.VMEM((128 * 128,), jnp.float32),
                  pltpu.VMEM((128 * 128,), jnp.float32),
                  pltpu.VMEM((128, 1, 128), jnp.float32),
                  pltpu.VMEM((128, 1, 128), jnp.float32))


_sc_scatter = functools.partial(
    pl.kernel,
    out_type=(jax.ShapeDtypeStruct((_BATCH * _SLAB + 128,), jnp.float32),
              jax.ShapeDtypeStruct((_SLAB, 8, 128), jnp.float32)),
    mesh=plsc.VectorSubcoreMesh(core_axis_name="c", subcore_axis_name="s"),
    compiler_params=pltpu.CompilerParams(needs_layout_passes=False),
    scratch_types=[
        pltpu.VMEM((2, _ROW_W), jnp.float32),
        pltpu.VMEM((112,), jnp.int32),
        pltpu.VMEM((112,), jnp.int32),
        pltpu.VMEM((_CHUNKS * 16,), jnp.int32),
        pltpu.VMEM((_CHUNKS * 16,), jnp.int32),
        pltpu.SemaphoreType.DMA,
        pltpu.SemaphoreType.DMA,
        pltpu.SemaphoreType.DMA,
        pltpu.SemaphoreType.DMA,
        pltpu.SemaphoreType.DMA,
    ],
)(_scatter_body)


@jax.jit
def kernel(inputs):
    _, out = _sc_scatter(inputs)
    t = out.reshape(_N_CHANNELS, _N_SAMPLES, 1, _BATCH)
    t = with_layout_constraint(
        t, Layout(major_to_minor=(0, 1, 2, 3), tiling=((1, 128),)))
    return jnp.transpose(t, (3, 0, 1, 2))


# speedup vs baseline: 1.3368x; 1.3368x over previous
import functools

import jax
import jax.numpy as jnp
from jax import lax
from jax.experimental import pallas as pl
from jax.experimental.layout import Layout, with_layout_constraint
from jax.experimental.pallas import tpu as pltpu
from jax.experimental.pallas import tpu_sc as plsc

_N_DENSE = 100
_N_SAMPLES = 20
_N_CHANNELS = 1000
_BATCH = 1024
_ROW_W = _N_DENSE + _N_DENSE * _N_SAMPLES
_SLAB = _N_CHANNELS * _N_SAMPLES

_NC = 2
_NS = 16
_NW = _NC * _NS
_ROWS_PER_W = _BATCH // _NW
_CHUNKS = _N_DENSE * _N_SAMPLES // 16
_NBLK = _SLAB // 128 + 1
_TAIL = _SLAB - (_NBLK - 1) * 128


def _scatter_body(inp_hbm, scr_hbm, out_hbm, inp_v, cidx0, cidx1, qv, rv,
                  sem0, sem1, semr, semw0, semw1):
    c = lax.axis_index("c")
    s = lax.axis_index("s")
    wid = c * _NS + s
    row0 = wid * _ROWS_PER_W

    zvec = jnp.zeros((16,), jnp.float32)
    iota = lax.iota(jnp.int32, 16)
    zvi = iota * 0

    for t in range(5):
        lo = t * 16
        bq = lo // _N_SAMPLES
        cross = (bq + 1) * _N_SAMPLES - lo
        qt = bq + jnp.where(iota >= cross, 1, 0)
        qv[pl.ds(lo, 16)] = qt
        rv[pl.ds(lo, 16)] = (lo + iota) - qt * _N_SAMPLES

    def _rep(j, carry):
        for t in range(5):
            src = pl.ds(t * 16, 16)
            dst = pl.ds(j * 80 + t * 16, 16)
            qv[dst] = qv[src] + j * 4
            rv[dst] = rv[src]
        return carry

    lax.fori_loop(1, _CHUNKS // 5, _rep, 0)

    def _phase1(acc0, acc1):
        def _zero(i, carry):
            dst = pl.ds(i * 16, 16)
            acc0[dst] = zvec
            acc1[dst] = zvec
            return carry

        lax.fori_loop(0, _SLAB // 16, _zero, 0)

        def _pair(p, carry):
            b = row0 + 2 * p
            pltpu.sync_copy(inp_hbm.at[pl.ds(b, 2)], inp_v)
            for r in (0, 1):
                acc = acc0 if r == 0 else acc1
                cidx = cidx0 if r == 0 else cidx1
                sem = sem0 if r == 0 else sem1
                rsp = zvi + r

                @pl.when(p > 0)
                def _():
                    pltpu.make_async_copy(
                        acc, scr_hbm.at[pl.ds(0, _SLAB)], sem).wait()

                    def _clear(k, cc):
                        ds16 = pl.ds(k * 16, 16)
                        fidx = (plsc.load_gather(cidx, [qv[ds16]])
                                + rv[ds16])
                        plsc.store_scatter(acc, [fidx], zvec)
                        return cc

                    lax.fori_loop(0, _CHUNKS, _clear, 0)

                for off in (0, 16, 32, 48, 64, 80, 96):
                    cidx[pl.ds(off, 16)] = (
                        plsc.load_gather(inp_v, [rsp, iota + off])
                        .astype(jnp.int32) * _N_SAMPLES)

                def _accum(k, cc):
                    ds16 = pl.ds(k * 16, 16)
                    fidx = (plsc.load_gather(cidx, [qv[ds16]])
                            + rv[ds16])
                    x = plsc.load_gather(
                        inp_v, [rsp, iota + (_N_DENSE + k * 16)])
                    plsc.addupdate_scatter(acc, [fidx], x)
                    return cc

                lax.fori_loop(0, _CHUNKS, _accum, 0)

                pltpu.async_copy(
                    acc, scr_hbm.at[pl.ds((b + r) * _SLAB, _SLAB)], sem)
            return carry

        lax.fori_loop(0, _ROWS_PER_W // 2, _pair, 0)

        pltpu.make_async_copy(acc0, scr_hbm.at[pl.ds(0, _SLAB)],
                              sem0).wait()
        pltpu.make_async_copy(acc1, scr_hbm.at[pl.ds(0, _SLAB)],
                              sem1).wait()

    pl.run_scoped(_phase1,
                  pltpu.VMEM((_SLAB,), jnp.float32),
                  pltpu.VMEM((_SLAB,), jnp.float32))

    plsc.subcore_barrier()

    iota128 = iota * 128

    def _phase2(in_a, in_b, out_a, out_b):
        def _fire(fv, jv, in_st):
            blkn = s * 10 + jv
            base = ((c * 4 + fv) * 128) * _SLAB + blkn * 128

            def _fr(i, cc):
                for k in range(16):
                    pltpu.async_copy(
                        scr_hbm.at[pl.ds(base + (i * 16 + k) * _SLAB,
                                         128)],
                        in_st.at[pl.ds((i * 16 + k) * 128, 128)],
                        semr)
                return cc

            lax.fori_loop(0, 8, _fr, 0)

        def _drain_reads(in_st):
            def _dr(i, cc):
                for k in range(16):
                    pltpu.make_async_copy(
                        scr_hbm.at[pl.ds(0, 128)],
                        in_st.at[pl.ds(0, 128)], semr).wait()
                return cc

            lax.fori_loop(0, 8, _dr, 0)

        def _unit(f, j, in_st, in_oth, out_st, semw, parity):
            blk = s * 10 + j
            u = f * 10 + j

            @pl.when(blk < _NBLK)
            def _():
                @pl.when(u >= 2)
                def _():
                    @pl.when(jnp.logical_or(j > 0, s < _NS - 1))
                    def _():
                        pltpu.make_async_copy(
                            out_st,
                            out_hbm.at[pl.ds(0, 128), pl.ds(0, 1), :],
                            semw).wait()

                    @pl.when(jnp.logical_and(j == 0, s == _NS - 1))
                    def _():
                        pltpu.make_async_copy(
                            out_st.at[pl.ds(0, _TAIL)],
                            out_hbm.at[pl.ds(0, _TAIL), pl.ds(0, 1), :],
                            semw).wait()

                _drain_reads(in_st)

                adv = jnp.logical_and(j < 9, blk + 1 < _NBLK)

                if parity == 0:
                    @pl.when(adv)
                    def _():
                        _fire(f, j + 1, in_oth)
                else:
                    fn = jnp.where(adv, f, f + 1)
                    jn = jnp.where(adv, j + 1, 0)

                    @pl.when(jnp.logical_or(adv, f < 3))
                    def _():
                        _fire(fn, jn, in_oth)

                def _tr(jo, cc):
                    for ji in range(8):
                        x = plsc.load_gather(
                            in_st, [iota128 + (ji * 2048 + jo)])
                        plsc.store_scatter(
                            out_st, [zvi + jo, zvi, iota + ji * 16], x)
                    return cc

                lax.fori_loop(0, 128, _tr, 0)

                if parity == 0:
                    @pl.when(jnp.logical_and(jnp.logical_not(adv), f < 3))
                    def _():
                        _fire(f + 1, 0, in_st)

                @pl.when(blk < _NBLK - 1)
                def _():
                    pltpu.async_copy(
                        out_st,
                        out_hbm.at[pl.ds(blk * 128, 128),
                                   pl.ds(c * 4 + f, 1), :],
                        semw)

                @pl.when(blk == _NBLK - 1)
                def _():
                    pltpu.async_copy(
                        out_st.at[pl.ds(0, _TAIL)],
                        out_hbm.at[pl.ds((_NBLK - 1) * 128, _TAIL),
                                   pl.ds(c * 4 + f, 1), :],
                        semw)

        _fire(0, 0, in_a)

        def _fbody(f, cc):
            def _jbody(j, cc2):
                @pl.when(lax.bitwise_and(j, 1) == 0)
                def _():
                    _unit(f, j, in_a, in_b, out_a, semw0, 0)

                @pl.when(lax.bitwise_and(j, 1) == 1)
                def _():
                    _unit(f, j, in_b, in_a, out_b, semw1, 1)
                return cc2

            return lax.fori_loop(0, 10, _jbody, cc)

        lax.fori_loop(0, 4, _fbody, 0)

        @pl.when(s < 15)
        def _():
            pltpu.make_async_copy(
                out_a, out_hbm.at[pl.ds(0, 128), pl.ds(0, 1), :],
                semw0).wait()
            pltpu.make_async_copy(
                out_b, out_hbm.at[pl.ds(0, 128), pl.ds(0, 1), :],
                semw1).wait()

        @pl.when(s == 15)
        def _():
            pltpu.make_async_copy(
                out_a.at[pl.ds(0, _TAIL)],
                out_hbm.at[pl.ds(0, _TAIL), pl.ds(0, 1), :],
                semw0).wait()
            pltpu.make_async_copy(
                out_b, out_hbm.at[pl.ds(0, 128), pl.ds(0, 1), :],
                semw1).wait()

    pl.run_scoped(_phase2,
                  pltpu.VMEM((128 * 128,), jnp.float32),
                  pltpu.VMEM((128 * 128,), jnp.float32),
                  pltpu.VMEM((128, 1, 128), jnp.float32),
                  pltpu.VMEM((128, 1, 128), jnp.float32))


_sc_scatter = functools.partial(
    pl.kernel,
    out_type=(jax.ShapeDtypeStruct((_BATCH * _SLAB + 128,), jnp.float32),
              jax.ShapeDtypeStruct((_SLAB, 8, 128), jnp.float32)),
    mesh=plsc.VectorSubcoreMesh(core_axis_name="c", subcore_axis_name="s"),
    compiler_params=pltpu.CompilerParams(needs_layout_passes=False),
    scratch_types=[
        pltpu.VMEM((2, _ROW_W), jnp.float32),
        pltpu.VMEM((112,), jnp.int32),
        pltpu.VMEM((112,), jnp.int32),
        pltpu.VMEM((_CHUNKS * 16,), jnp.int32),
        pltpu.VMEM((_CHUNKS * 16,), jnp.int32),
        pltpu.SemaphoreType.DMA,
        pltpu.SemaphoreType.DMA,
        pltpu.SemaphoreType.DMA,
        pltpu.SemaphoreType.DMA,
        pltpu.SemaphoreType.DMA,
    ],
)(_scatter_body)


@jax.jit
def kernel(inputs):
    _, out = _sc_scatter(inputs)
    t = out.reshape(_N_CHANNELS, _N_SAMPLES, 1, _BATCH)
    t = with_layout_constraint(
        t, Layout(major_to_minor=(0, 1, 2, 3), tiling=((1, 128),)))
    return jnp.transpose(t, (3, 0, 1, 2))


# speedup vs baseline: 1.3778x; 1.0306x over previous
import functools

import jax
import jax.numpy as jnp
from jax import lax
from jax.experimental import pallas as pl
from jax.experimental.layout import Layout, with_layout_constraint
from jax.experimental.pallas import tpu as pltpu
from jax.experimental.pallas import tpu_sc as plsc

_N_DENSE = 100
_N_SAMPLES = 20
_N_CHANNELS = 1000
_BATCH = 1024
_ROW_W = _N_DENSE + _N_DENSE * _N_SAMPLES
_SLAB = _N_CHANNELS * _N_SAMPLES

_NC = 2
_NS = 16
_NW = _NC * _NS
_ROWS_PER_W = _BATCH // _NW
_CHUNKS = _N_DENSE * _N_SAMPLES // 16
_NBLK = _SLAB // 128 + 1
_TAIL = _SLAB - (_NBLK - 1) * 128


def _scatter_body(inp_hbm, scr_hbm, out_hbm, inp_a2, inp_b2, cidx0, cidx1,
                  qv, rv, sem0, sem1, semr, semw0, semw1, semi0, semi1):
    c = lax.axis_index("c")
    s = lax.axis_index("s")
    wid = c * _NS + s
    row0 = wid * _ROWS_PER_W

    zvec = jnp.zeros((16,), jnp.float32)
    iota = lax.iota(jnp.int32, 16)
    zvi = iota * 0

    for t in range(5):
        lo = t * 16
        bq = lo // _N_SAMPLES
        cross = (bq + 1) * _N_SAMPLES - lo
        qt = bq + jnp.where(iota >= cross, 1, 0)
        qv[pl.ds(lo, 16)] = qt
        rv[pl.ds(lo, 16)] = (lo + iota) - qt * _N_SAMPLES

    def _rep(j, carry):
        for t in range(5):
            src = pl.ds(t * 16, 16)
            dst = pl.ds(j * 80 + t * 16, 16)
            qv[dst] = qv[src] + j * 4
            rv[dst] = rv[src]
        return carry

    lax.fori_loop(1, _CHUNKS // 5, _rep, 0)

    def _phase1(acc0, acc1):
        def _zero(i, carry):
            dst = pl.ds(i * 16, 16)
            acc0[dst] = zvec
            acc1[dst] = zvec
            return carry

        lax.fori_loop(0, _SLAB // 16, _zero, 0)

        pltpu.async_copy(inp_hbm.at[pl.ds(row0, 2)], inp_a2, semi0)

        def _pair(p, carry):
            b = row0 + 2 * p
            par = lax.bitwise_and(p, 1)

            @pl.when(par == 0)
            def _():
                pltpu.make_async_copy(inp_hbm.at[pl.ds(0, 2)], inp_a2,
                                      semi0).wait()

                @pl.when(p + 1 < _ROWS_PER_W // 2)
                def _():
                    pltpu.async_copy(inp_hbm.at[pl.ds(b + 2, 2)], inp_b2,
                                     semi1)
                _rows(p, inp_a2)

            @pl.when(par == 1)
            def _():
                pltpu.make_async_copy(inp_hbm.at[pl.ds(0, 2)], inp_b2,
                                      semi1).wait()

                @pl.when(p + 1 < _ROWS_PER_W // 2)
                def _():
                    pltpu.async_copy(inp_hbm.at[pl.ds(b + 2, 2)], inp_a2,
                                     semi0)
                _rows(p, inp_b2)
            return carry

        def _rows(p, inp_v):
            b = row0 + 2 * p
            for r in (0, 1):
                acc = acc0 if r == 0 else acc1
                cidx = cidx0 if r == 0 else cidx1
                sem = sem0 if r == 0 else sem1
                rsp = zvi + r

                @pl.when(p > 0)
                def _():
                    pltpu.make_async_copy(
                        acc, scr_hbm.at[pl.ds(0, _SLAB)], sem).wait()

                    def _clear(k, cc):
                        ds16 = pl.ds(k * 16, 16)
                        fidx = (plsc.load_gather(cidx, [qv[ds16]])
                                + rv[ds16])
                        plsc.store_scatter(acc, [fidx], zvec)
                        return cc

                    lax.fori_loop(0, _CHUNKS, _clear, 0)

                for off in (0, 16, 32, 48, 64, 80, 96):
                    cidx[pl.ds(off, 16)] = (
                        plsc.load_gather(inp_v, [rsp, iota + off])
                        .astype(jnp.int32) * _N_SAMPLES)

                def _accum(k, cc):
                    ds16 = pl.ds(k * 16, 16)
                    fidx = (plsc.load_gather(cidx, [qv[ds16]])
                            + rv[ds16])
                    x = plsc.load_gather(
                        inp_v, [rsp, iota + (_N_DENSE + k * 16)])
                    plsc.addupdate_scatter(acc, [fidx], x)
                    return cc

                lax.fori_loop(0, _CHUNKS, _accum, 0)

                pltpu.async_copy(
                    acc, scr_hbm.at[pl.ds((b + r) * _SLAB, _SLAB)], sem)

        lax.fori_loop(0, _ROWS_PER_W // 2, _pair, 0)

        pltpu.make_async_copy(acc0, scr_hbm.at[pl.ds(0, _SLAB)],
                              sem0).wait()
        pltpu.make_async_copy(acc1, scr_hbm.at[pl.ds(0, _SLAB)],
                              sem1).wait()

    pl.run_scoped(_phase1,
                  pltpu.VMEM((_SLAB,), jnp.float32),
                  pltpu.VMEM((_SLAB,), jnp.float32))

    plsc.subcore_barrier()

    iota128 = iota * 128

    def _phase2(in_a, in_b, out_a, out_b):
        def _fire(fv, jv, in_st):
            blkn = s * 10 + jv
            base = ((c * 4 + fv) * 128) * _SLAB + blkn * 128

            def _fr(i, cc):
                for k in range(16):
                    pltpu.async_copy(
                        scr_hbm.at[pl.ds(base + (i * 16 + k) * _SLAB,
                                         128)],
                        in_st.at[pl.ds((i * 16 + k) * 128, 128)],
                        semr)
                return cc

            lax.fori_loop(0, 8, _fr, 0)

        def _drain_reads(in_st):
            def _dr(i, cc):
                for k in range(16):
                    pltpu.make_async_copy(
                        scr_hbm.at[pl.ds(0, 128)],
                        in_st.at[pl.ds(0, 128)], semr).wait()
                return cc

            lax.fori_loop(0, 8, _dr, 0)

        def _unit(f, j, in_st, in_oth, out_st, semw, parity):
            blk = s * 10 + j
            u = f * 10 + j

            @pl.when(blk < _NBLK)
            def _():
                @pl.when(u >= 2)
                def _():
                    @pl.when(jnp.logical_or(j > 0, s < _NS - 1))
                    def _():
                        pltpu.make_async_copy(
                            out_st,
                            out_hbm.at[pl.ds(0, 128), pl.ds(0, 1), :],
                            semw).wait()

                    @pl.when(jnp.logical_and(j == 0, s == _NS - 1))
                    def _():
                        pltpu.make_async_copy(
                            out_st.at[pl.ds(0, _TAIL)],
                            out_hbm.at[pl.ds(0, _TAIL), pl.ds(0, 1), :],
                            semw).wait()

                _drain_reads(in_st)

                adv = jnp.logical_and(j < 9, blk + 1 < _NBLK)

                if parity == 0:
                    @pl.when(adv)
                    def _():
                        _fire(f, j + 1, in_oth)
                else:
                    fn = jnp.where(adv, f, f + 1)
                    jn = jnp.where(adv, j + 1, 0)

                    @pl.when(jnp.logical_or(adv, f < 3))
                    def _():
                        _fire(fn, jn, in_oth)

                def _tr(jo, cc):
                    for ji in range(8):
                        x = plsc.load_gather(
                            in_st, [iota128 + (ji * 2048 + jo)])
                        out_st[jo, 0, pl.ds(ji * 16, 16)] = x
                    return cc

                lax.fori_loop(0, 128, _tr, 0)

                if parity == 0:
                    @pl.when(jnp.logical_and(jnp.logical_not(adv), f < 3))
                    def _():
                        _fire(f + 1, 0, in_st)

                @pl.when(blk < _NBLK - 1)
                def _():
                    pltpu.async_copy(
                        out_st,
                        out_hbm.at[pl.ds(blk * 128, 128),
                                   pl.ds(c * 4 + f, 1), :],
                        semw)

                @pl.when(blk == _NBLK - 1)
                def _():
                    pltpu.async_copy(
                        out_st.at[pl.ds(0, _TAIL)],
                        out_hbm.at[pl.ds((_NBLK - 1) * 128, _TAIL),
                                   pl.ds(c * 4 + f, 1), :],
                        semw)

        _fire(0, 0, in_a)

        def _fbody(f, cc):
            def _jbody(j, cc2):
                @pl.when(lax.bitwise_and(j, 1) == 0)
                def _():
                    _unit(f, j, in_a, in_b, out_a, semw0, 0)

                @pl.when(lax.bitwise_and(j, 1) == 1)
                def _():
                    _unit(f, j, in_b, in_a, out_b, semw1, 1)
                return cc2

            return lax.fori_loop(0, 10, _jbody, cc)

        lax.fori_loop(0, 4, _fbody, 0)

        @pl.when(s < 15)
        def _():
            pltpu.make_async_copy(
                out_a, out_hbm.at[pl.ds(0, 128), pl.ds(0, 1), :],
                semw0).wait()
            pltpu.make_async_copy(
                out_b, out_hbm.at[pl.ds(0, 128), pl.ds(0, 1), :],
                semw1).wait()

        @pl.when(s == 15)
        def _():
            pltpu.make_async_copy(
                out_a.at[pl.ds(0, _TAIL)],
                out_hbm.at[pl.ds(0, _TAIL), pl.ds(0, 1), :],
                semw0).wait()
            pltpu.make_async_copy(
                out_b, out_hbm.at[pl.ds(0, 128), pl.ds(0, 1), :],
                semw1).wait()

    pl.run_scoped(_phase2,
                  pltpu.VMEM((128 * 128,), jnp.float32),
                  pltpu.VMEM((128 * 128,), jnp.float32),
                  pltpu.VMEM((128, 1, 128), jnp.float32),
                  pltpu.VMEM((128, 1, 128), jnp.float32))


_sc_scatter = functools.partial(
    pl.kernel,
    out_type=(jax.ShapeDtypeStruct((_BATCH * _SLAB + 128,), jnp.float32),
              jax.ShapeDtypeStruct((_SLAB, 8, 128), jnp.float32)),
    mesh=plsc.VectorSubcoreMesh(core_axis_name="c", subcore_axis_name="s"),
    compiler_params=pltpu.CompilerParams(needs_layout_passes=False),
    scratch_types=[
        pltpu.VMEM((2, _ROW_W), jnp.float32),
        pltpu.VMEM((2, _ROW_W), jnp.float32),
        pltpu.VMEM((112,), jnp.int32),
        pltpu.VMEM((112,), jnp.int32),
        pltpu.VMEM((_CHUNKS * 16,), jnp.int32),
        pltpu.VMEM((_CHUNKS * 16,), jnp.int32),
        pltpu.SemaphoreType.DMA,
        pltpu.SemaphoreType.DMA,
        pltpu.SemaphoreType.DMA,
        pltpu.SemaphoreType.DMA,
        pltpu.SemaphoreType.DMA,
        pltpu.SemaphoreType.DMA,
        pltpu.SemaphoreType.DMA,
    ],
)(_scatter_body)


@jax.jit
def kernel(inputs):
    _, out = _sc_scatter(inputs)
    t = out.reshape(_N_CHANNELS, _N_SAMPLES, 1, _BATCH)
    t = with_layout_constraint(
        t, Layout(major_to_minor=(0, 1, 2, 3), tiling=((1, 128),)))
    return jnp.transpose(t, (3, 0, 1, 2))


# speedup vs baseline: 1.4611x; 1.0605x over previous
import functools

import jax
import jax.numpy as jnp
from jax import lax
from jax.experimental import pallas as pl
from jax.experimental.layout import Layout, with_layout_constraint
from jax.experimental.pallas import tpu as pltpu
from jax.experimental.pallas import tpu_sc as plsc

_N_DENSE = 100
_N_SAMPLES = 20
_N_CHANNELS = 1000
_BATCH = 1024
_ROW_W = _N_DENSE + _N_DENSE * _N_SAMPLES
_SLAB = _N_CHANNELS * _N_SAMPLES

_NC = 2
_NS = 16
_NW = _NC * _NS
_ROWS_PER_W = _BATCH // _NW
_CHUNKS = _N_DENSE * _N_SAMPLES // 16
_NBLK = _SLAB // 128 + 1
_NROW = _NBLK
_TAIL = _SLAB - (_NBLK - 1) * 128


def _scatter_body(inp_hbm, scr_hbm, out_hbm, inp_a2, inp_b2, cidx0, cidx1,
                  qv, rv, idxv, sem0, sem1, semr, semw0, semw1, semi0,
                  semi1):
    c = lax.axis_index("c")
    s = lax.axis_index("s")
    wid = c * _NS + s
    row0 = wid * _ROWS_PER_W

    zvec = jnp.zeros((16,), jnp.float32)
    iota = lax.iota(jnp.int32, 16)
    zvi = iota * 0

    for t in range(5):
        lo = t * 16
        bq = lo // _N_SAMPLES
        cross = (bq + 1) * _N_SAMPLES - lo
        qt = bq + jnp.where(iota >= cross, 1, 0)
        qv[pl.ds(lo, 16)] = qt
        rv[pl.ds(lo, 16)] = (lo + iota) - qt * _N_SAMPLES

    def _rep(j, carry):
        for t in range(5):
            src = pl.ds(t * 16, 16)
            dst = pl.ds(j * 80 + t * 16, 16)
            qv[dst] = qv[src] + j * 4
            rv[dst] = rv[src]
        return carry

    lax.fori_loop(1, _CHUNKS // 5, _rep, 0)

    def _phase1(acc0, acc1):
        def _zero(i, carry):
            for k in range(8):
                dst = pl.ds(k * 16, 16)
                acc0[i, 0, dst] = zvec
                acc1[i, 0, dst] = zvec
            return carry

        lax.fori_loop(0, _NROW, _zero, 0)

        pltpu.async_copy(inp_hbm.at[pl.ds(row0, 2)], inp_a2, semi0)

        def _pair(p, carry):
            b = row0 + 2 * p
            par = lax.bitwise_and(p, 1)

            @pl.when(par == 0)
            def _():
                pltpu.make_async_copy(inp_hbm.at[pl.ds(0, 2)], inp_a2,
                                      semi0).wait()

                @pl.when(p + 1 < _ROWS_PER_W // 2)
                def _():
                    pltpu.async_copy(inp_hbm.at[pl.ds(b + 2, 2)], inp_b2,
                                     semi1)
                _rows(p, inp_a2)

            @pl.when(par == 1)
            def _():
                pltpu.make_async_copy(inp_hbm.at[pl.ds(0, 2)], inp_b2,
                                      semi1).wait()

                @pl.when(p + 1 < _ROWS_PER_W // 2)
                def _():
                    pltpu.async_copy(inp_hbm.at[pl.ds(b + 2, 2)], inp_a2,
                                     semi0)
                _rows(p, inp_b2)
            return carry

        def _rows(p, inp_v):
            b = row0 + 2 * p
            for r in (0, 1):
                acc = acc0 if r == 0 else acc1
                cidx = cidx0 if r == 0 else cidx1
                sem = sem0 if r == 0 else sem1
                rsp = zvi + r

                @pl.when(p > 0)
                def _():
                    pltpu.make_async_copy(
                        acc, scr_hbm.at[pl.ds(0, _NROW)], sem).wait()

                    def _clear(k, cc):
                        ds16 = pl.ds(k * 16, 16)
                        fidx = (plsc.load_gather(cidx, [qv[ds16]])
                                + rv[ds16])
                        plsc.store_scatter(
                            acc,
                            [lax.shift_right_logical(fidx, 7), zvi,
                             lax.bitwise_and(fidx, 127)], zvec)
                        return cc

                    lax.fori_loop(0, _CHUNKS, _clear, 0)

                for off in (0, 16, 32, 48, 64, 80, 96):
                    cidx[pl.ds(off, 16)] = (
                        plsc.load_gather(inp_v, [rsp, iota + off])
                        .astype(jnp.int32) * _N_SAMPLES)

                def _accum(k, cc):
                    ds16 = pl.ds(k * 16, 16)
                    fidx = (plsc.load_gather(cidx, [qv[ds16]])
                            + rv[ds16])
                    x = plsc.load_gather(
                        inp_v, [rsp, iota + (_N_DENSE + k * 16)])
                    plsc.addupdate_scatter(
                        acc,
                        [lax.shift_right_logical(fidx, 7), zvi,
                         lax.bitwise_and(fidx, 127)], x)
                    return cc

                lax.fori_loop(0, _CHUNKS, _accum, 0)

                pltpu.async_copy(
                    acc, scr_hbm.at[pl.ds((b + r) * _NROW, _NROW)], sem)

        lax.fori_loop(0, _ROWS_PER_W // 2, _pair, 0)

        pltpu.make_async_copy(acc0, scr_hbm.at[pl.ds(0, _NROW)],
                              sem0).wait()
        pltpu.make_async_copy(acc1, scr_hbm.at[pl.ds(0, _NROW)],
                              sem1).wait()

    pl.run_scoped(_phase1,
                  pltpu.VMEM((_NROW, 1, 128), jnp.float32),
                  pltpu.VMEM((_NROW, 1, 128), jnp.float32))

    plsc.subcore_barrier()

    iota128 = iota * 128

    def _phase2(in_a, in_b, out_a, out_b):
        def _fire(fv, jv, in_st):
            blkn = s * 10 + jv
            bb = (c * 4 + fv) * 128
            for k in range(8):
                idxv[pl.ds(k * 16, 16)] = (
                    (iota + (bb + k * 16)) * _NROW + blkn)
            pltpu.async_copy(scr_hbm.at[idxv], in_st, semr)

        def _drain_reads(in_st):
            pltpu.make_async_copy(scr_hbm.at[idxv], in_st, semr).wait()

        def _unit(f, j, in_st, in_oth, out_st, semw, parity):
            blk = s * 10 + j
            u = f * 10 + j

            @pl.when(blk < _NBLK)
            def _():
                @pl.when(u >= 2)
                def _():
                    @pl.when(jnp.logical_or(j > 0, s < _NS - 1))
                    def _():
                        pltpu.make_async_copy(
                            out_st,
                            out_hbm.at[pl.ds(0, 128), pl.ds(0, 1), :],
                            semw).wait()

                    @pl.when(jnp.logical_and(j == 0, s == _NS - 1))
                    def _():
                        pltpu.make_async_copy(
                            out_st.at[pl.ds(0, _TAIL)],
                            out_hbm.at[pl.ds(0, _TAIL), pl.ds(0, 1), :],
                            semw).wait()

                _drain_reads(in_st)

                adv = jnp.logical_and(j < 9, blk + 1 < _NBLK)

                if parity == 0:
                    @pl.when(adv)
                    def _():
                        _fire(f, j + 1, in_oth)
                else:
                    fn = jnp.where(adv, f, f + 1)
                    jn = jnp.where(adv, j + 1, 0)

                    @pl.when(jnp.logical_or(adv, f < 3))
                    def _():
                        _fire(fn, jn, in_oth)

                def _tr(jo, cc):
                    jos = zvi + jo
                    for ji in range(8):
                        x = plsc.load_gather(
                            in_st, [iota + ji * 16, zvi, jos])
                        out_st[jo, 0, pl.ds(ji * 16, 16)] = x
                    return cc

                lax.fori_loop(0, 128, _tr, 0)

                if parity == 0:
                    @pl.when(jnp.logical_and(jnp.logical_not(adv), f < 3))
                    def _():
                        _fire(f + 1, 0, in_st)

                @pl.when(blk < _NBLK - 1)
                def _():
                    pltpu.async_copy(
                        out_st,
                        out_hbm.at[pl.ds(blk * 128, 128),
                                   pl.ds(c * 4 + f, 1), :],
                        semw)

                @pl.when(blk == _NBLK - 1)
                def _():
                    pltpu.async_copy(
                        out_st.at[pl.ds(0, _TAIL)],
                        out_hbm.at[pl.ds((_NBLK - 1) * 128, _TAIL),
                                   pl.ds(c * 4 + f, 1), :],
                        semw)

        _fire(0, 0, in_a)

        def _fbody(f, cc):
            def _jbody(j, cc2):
                @pl.when(lax.bitwise_and(j, 1) == 0)
                def _():
                    _unit(f, j, in_a, in_b, out_a, semw0, 0)

                @pl.when(lax.bitwise_and(j, 1) == 1)
                def _():
                    _unit(f, j, in_b, in_a, out_b, semw1, 1)
                return cc2

            return lax.fori_loop(0, 10, _jbody, cc)

        lax.fori_loop(0, 4, _fbody, 0)

        @pl.when(s < 15)
        def _():
            pltpu.make_async_copy(
                out_a, out_hbm.at[pl.ds(0, 128), pl.ds(0, 1), :],
                semw0).wait()
            pltpu.make_async_copy(
                out_b, out_hbm.at[pl.ds(0, 128), pl.ds(0, 1), :],
                semw1).wait()

        @pl.when(s == 15)
        def _():
            pltpu.make_async_copy(
                out_a.at[pl.ds(0, _TAIL)],
                out_hbm.at[pl.ds(0, _TAIL), pl.ds(0, 1), :],
                semw0).wait()
            pltpu.make_async_copy(
                out_b, out_hbm.at[pl.ds(0, 128), pl.ds(0, 1), :],
                semw1).wait()

    pl.run_scoped(_phase2,
                  pltpu.VMEM((128, 1, 128), jnp.float32),
                  pltpu.VMEM((128, 1, 128), jnp.float32),
                  pltpu.VMEM((128, 1, 128), jnp.float32),
                  pltpu.VMEM((128, 1, 128), jnp.float32))


_sc_scatter = functools.partial(
    pl.kernel,
    out_type=(jax.ShapeDtypeStruct((_BATCH * _NROW, 1, 128), jnp.float32),
              jax.ShapeDtypeStruct((_SLAB, 8, 128), jnp.float32)),
    mesh=plsc.VectorSubcoreMesh(core_axis_name="c", subcore_axis_name="s"),
    compiler_params=pltpu.CompilerParams(needs_layout_passes=False),
    scratch_types=[
        pltpu.VMEM((2, _ROW_W), jnp.float32),
        pltpu.VMEM((2, _ROW_W), jnp.float32),
        pltpu.VMEM((112,), jnp.int32),
        pltpu.VMEM((112,), jnp.int32),
        pltpu.VMEM((_CHUNKS * 16,), jnp.int32),
        pltpu.VMEM((_CHUNKS * 16,), jnp.int32),
        pltpu.VMEM((128,), jnp.int32),
        pltpu.SemaphoreType.DMA,
        pltpu.SemaphoreType.DMA,
        pltpu.SemaphoreType.DMA,
        pltpu.SemaphoreType.DMA,
        pltpu.SemaphoreType.DMA,
        pltpu.SemaphoreType.DMA,
        pltpu.SemaphoreType.DMA,
    ],
)(_scatter_body)


@jax.jit
def kernel(inputs):
    _, out = _sc_scatter(inputs)
    t = out.reshape(_N_CHANNELS, _N_SAMPLES, 1, _BATCH)
    t = with_layout_constraint(
        t, Layout(major_to_minor=(0, 1, 2, 3), tiling=((1, 128),)))
    return jnp.transpose(t, (3, 0, 1, 2))
